# baseline (speedup 1.0000x reference)
import functools

import jax
import jax.numpy as jnp
import numpy as np
from jax import lax
from jax.experimental import pallas as pl
from jax.experimental.pallas import tpu as pltpu
from jax.experimental.pallas import tpu_sc as plsc

_HALF_PI = float(np.pi / 2)


def _freq_phase(dim):
    i = np.arange(dim, dtype=np.float64)
    inv_freq = np.power(10000.0, -(i - (i % 2)) / dim)
    phase = (i % 2) * (np.pi / 2)
    return (jnp.asarray(inv_freq, dtype=jnp.float32),
            jnp.asarray(phase, dtype=jnp.float32))


def _sc_broadcast_rows(table, nb, t, dim):
    info = plsc.get_sparse_core_info()
    nw = info.num_cores * info.num_subcores
    rows_per_w = t // nw
    chunk = min(rows_per_w, 32)
    n_chunks = rows_per_w // chunk
    mesh = plsc.VectorSubcoreMesh(core_axis_name="c", subcore_axis_name="s")

    @functools.partial(
        pl.kernel,
        mesh=mesh,
        out_type=jax.ShapeDtypeStruct((nb * t, dim), jnp.float32),
        scratch_types=[
            pltpu.VMEM((2, chunk, dim), jnp.float32),
            pltpu.SemaphoreType.DMA,
            pltpu.SemaphoreType.DMA,
            pltpu.SemaphoreType.DMA,
        ],
    )
    def k(table_hbm, out_hbm, buf, ld_sem, st_sem0, st_sem1):
        wid = lax.axis_index("s") * info.num_cores + lax.axis_index("c")
        base = wid * rows_per_w
        st_sems = (st_sem0, st_sem1)

        def start_load(c):
            return pltpu.async_copy(
                table_hbm.at[pl.ds(base + c * chunk, chunk)], buf.at[c % 2], ld_sem
            )

        loads = [None] * n_chunks
        stores = [[] for _ in range(n_chunks)]
        loads[0] = start_load(0)
        for c in range(n_chunks):
            loads[c].wait()
            if c + 1 < n_chunks:
                if c >= 1:
                    for d in stores[c - 1]:
                        d.wait()
                loads[c + 1] = start_load(c + 1)
            row0 = base + c * chunk
            for bb in range(nb):
                stores[c].append(
                    pltpu.async_copy(
                        buf.at[c % 2],
                        out_hbm.at[pl.ds(bb * t + row0, chunk)],
                        st_sems[c % 2],
                    )
                )
        for c in (n_chunks - 2, n_chunks - 1):
            if c >= 0:
                for d in stores[c]:
                    d.wait()

    return k(table).reshape(nb, t, dim)


def _tc_compute_rows(inv_freq, phase, nb, t, dim):
    bt = 512
    grid = (t // bt,)

    def body(invf_ref, ph_ref, out_ref):
        i = pl.program_id(0)
        pos = (
            jax.lax.broadcasted_iota(jnp.int32, (bt, dim), 0) + i * bt
        ).astype(jnp.float32)
        val = jnp.sin(pos * invf_ref[...] + ph_ref[...])
        for bb in range(nb):
            out_ref[bb] = val

    return pl.pallas_call(
        body,
        grid=grid,
        in_specs=[
            pl.BlockSpec((1, dim), lambda i: (0, 0)),
            pl.BlockSpec((1, dim), lambda i: (0, 0)),
        ],
        out_specs=pl.BlockSpec((nb, bt, dim), lambda i: (0, i, 0)),
        out_shape=jax.ShapeDtypeStruct((nb, t, dim), jnp.float32),
    )(inv_freq.reshape(1, dim), phase.reshape(1, dim))


def kernel(inputs):
    b, t, dim = inputs.shape
    inv_freq, phase = _freq_phase(dim)
    nb_sc = b // 2
    nb_tc = b - nb_sc
    pos = jax.lax.broadcasted_iota(jnp.float32, (t, dim), 0)
    table = jnp.sin(pos * inv_freq[None, :] + phase[None, :])
    sc_part = _sc_broadcast_rows(table, nb_sc, t, dim)
    tc_part = _tc_compute_rows(inv_freq, phase, nb_tc, t, dim)
    return jnp.concatenate([sc_part, tc_part], axis=0)


# speedup vs baseline: 3.1464x; 3.1464x over previous
import functools

import jax
import jax.numpy as jnp
import numpy as np
from jax import lax
from jax.experimental import pallas as pl
from jax.experimental.pallas import tpu as pltpu
from jax.experimental.pallas import tpu_sc as plsc


def _table_factors(t, dim, split):
    i = np.arange(dim, dtype=np.float64)
    w = np.power(10000.0, -(i - (i % 2)) / dim)
    ph = (i % 2) * (np.pi / 2)
    h = np.arange(t // split, dtype=np.float64)[:, None]
    l = np.arange(split, dtype=np.float64)[:, None]
    hs = np.sin(h * split * w)
    hc = np.cos(h * split * w)
    ls = np.sin(l * w + ph)
    lc = np.cos(l * w + ph)
    return tuple(jnp.asarray(x, dtype=jnp.float32) for x in (hs, hc, ls, lc))


def _sc_broadcast_rows(table, nb, t, dim):
    info = plsc.get_sparse_core_info()
    nw = info.num_cores * info.num_subcores
    rows_per_w = t // nw
    chunk = min(rows_per_w, 32)
    n_chunks = rows_per_w // chunk
    mesh = plsc.VectorSubcoreMesh(core_axis_name="c", subcore_axis_name="s")

    @functools.partial(
        pl.kernel,
        mesh=mesh,
        out_type=jax.ShapeDtypeStruct((nb * t, dim), jnp.float32),
        scratch_types=[
            pltpu.VMEM((2, chunk, dim), jnp.float32),
            pltpu.SemaphoreType.DMA,
            pltpu.SemaphoreType.DMA,
            pltpu.SemaphoreType.DMA,
        ],
    )
    def k(table_hbm, out_hbm, buf, ld_sem, st_sem0, st_sem1):
        wid = lax.axis_index("s") * info.num_cores + lax.axis_index("c")
        base = wid * rows_per_w
        st_sems = (st_sem0, st_sem1)

        def start_load(c):
            return pltpu.async_copy(
                table_hbm.at[pl.ds(base + c * chunk, chunk)], buf.at[c % 2], ld_sem
            )

        loads = [None] * n_chunks
        stores = [[] for _ in range(n_chunks)]
        loads[0] = start_load(0)
        for c in range(n_chunks):
            loads[c].wait()
            if c + 1 < n_chunks:
                if c >= 1:
                    for d in stores[c - 1]:
                        d.wait()
                loads[c + 1] = start_load(c + 1)
            row0 = base + c * chunk
            for bb in range(nb):
                stores[c].append(
                    pltpu.async_copy(
                        buf.at[c % 2],
                        out_hbm.at[pl.ds(bb * t + row0, chunk)],
                        st_sems[c % 2],
                    )
                )
        for c in (n_chunks - 2, n_chunks - 1):
            if c >= 0:
                for d in stores[c]:
                    d.wait()

    return k(table).reshape(nb, t, dim)


def kernel(inputs):
    b, t, dim = inputs.shape
    split = 64
    hs, hc, ls, lc = _table_factors(t, dim, split)
    guard = jnp.float32(0.0) * inputs[0, 0, 0]
    table = (
        hs[:, None, :] * lc[None, :, :] + hc[:, None, :] * ls[None, :, :] + guard
    ).reshape(t, dim)
    return _sc_broadcast_rows(table, b, t, dim)
